# Initial kernel scaffold; baseline (speedup 1.0000x reference)
import jax
import jax.numpy as jnp
from jax.experimental import pallas as pl


def kernel(H, edge_index, dist2, delta, edge_struct, eW1, eb1, eW2, eb2, gW1, gb1, gW2, gb2, nW1, nb1, nW2, nb2, ln_g, ln_b):
    raise NotImplementedError("write your pallas kernel here")


# speedup vs baseline: 2.6286x; 2.6286x over previous
import functools

import jax
import jax.numpy as jnp
from jax import lax
from jax.experimental import pallas as pl
from jax.experimental.pallas import tpu as pltpu
from jax.experimental.pallas import tpu_sc as plsc

F32 = jnp.float32

_NC, _NS = 2, 16
_NW = _NC * _NS
_CH = 128


def _sigmoid(x):
    return 1.0 / (1.0 + jnp.exp(-x))


def _silu(x):
    return x * _sigmoid(x)


def _proj_body(h_ref, wi_ref, wj_ref, a_ref, b_ref):
    h = h_ref[...]
    a_ref[...] = jnp.dot(h, wi_ref[...], preferred_element_type=F32)
    b_ref[...] = jnp.dot(h, wj_ref[...], preferred_element_type=F32)


def _proj(h, wi, wj):
    n, d = h.shape
    blk = 2000
    return pl.pallas_call(
        _proj_body,
        grid=(n // blk,),
        in_specs=[
            pl.BlockSpec((blk, d), lambda i: (i, 0)),
            pl.BlockSpec((d, 64), lambda i: (0, 0)),
            pl.BlockSpec((d, 64), lambda i: (0, 0)),
        ],
        out_specs=[
            pl.BlockSpec((blk, 64), lambda i: (i, 0)),
            pl.BlockSpec((blk, 64), lambda i: (i, 0)),
        ],
        out_shape=[
            jax.ShapeDtypeStruct((n, 64), F32),
            jax.ShapeDtypeStruct((n, 64), F32),
        ],
    )(h, wi, wj)


def _gather_body(e, a_hbm, b_hbm, ii_hbm, jj_hbm, oa_hbm, ob_hbm,
                 ii_v, jj_v, rows_a, rows_b, sem_a, sem_b):
    c = lax.axis_index("c")
    s = lax.axis_index("s")
    wid = s * _NC + c
    ep = e // _NW
    nfull = ep // _CH
    tail = ep - nfull * _CH
    base = pl.multiple_of(wid * ep, _CH)

    pltpu.sync_copy(ii_hbm.at[pl.ds(base, ep)], ii_v)
    pltpu.sync_copy(jj_hbm.at[pl.ds(base, ep)], jj_v)

    def chunk(k, _):
        off = pl.multiple_of(k * _CH, _CH)
        ca = pltpu.async_copy(a_hbm.at[ii_v.at[pl.ds(off, _CH)]], rows_a, sem_a)
        cb = pltpu.async_copy(b_hbm.at[jj_v.at[pl.ds(off, _CH)]], rows_b, sem_b)
        ca.wait()
        cb.wait()
        pltpu.sync_copy(rows_a, oa_hbm.at[pl.ds(base + off, _CH)])
        pltpu.sync_copy(rows_b, ob_hbm.at[pl.ds(base + off, _CH)])
        return 0

    lax.fori_loop(0, nfull, chunk, 0)

    if tail:
        off = nfull * _CH
        ca = pltpu.async_copy(a_hbm.at[ii_v.at[pl.ds(off, tail)]],
                              rows_a.at[pl.ds(0, tail)], sem_a)
        cb = pltpu.async_copy(b_hbm.at[jj_v.at[pl.ds(off, tail)]],
                              rows_b.at[pl.ds(0, tail)], sem_b)
        ca.wait()
        cb.wait()
        pltpu.sync_copy(rows_a.at[pl.ds(0, tail)], oa_hbm.at[pl.ds(base + off, tail)])
        pltpu.sync_copy(rows_b.at[pl.ds(0, tail)], ob_hbm.at[pl.ds(base + off, tail)])


def _gather(a, bm, ii, jj):
    n, d = a.shape
    e = ii.shape[0]
    ep = e // _NW
    mesh = plsc.VectorSubcoreMesh(core_axis_name="c", subcore_axis_name="s",
                                  num_cores=_NC, num_subcores=_NS)
    k = pl.kernel(
        functools.partial(_gather_body, e),
        mesh=mesh,
        compiler_params=pltpu.CompilerParams(use_tc_tiling_on_sc=False),
        out_type=[
            jax.ShapeDtypeStruct((e, d), F32),
            jax.ShapeDtypeStruct((e, d), F32),
        ],
        scratch_types=[
            pltpu.VMEM((ep,), jnp.int32),
            pltpu.VMEM((ep,), jnp.int32),
            pltpu.VMEM((_CH, d), F32),
            pltpu.VMEM((_CH, d), F32),
            pltpu.SemaphoreType.DMA,
            pltpu.SemaphoreType.DMA,
        ],
    )
    return k(a, bm, ii, jj)


def _edge_body(ag_ref, bg_ref, d2_ref, dl_ref, st_ref,
               w1g_ref, eb1_ref, ew2_ref, eb2_ref,
               gw1_ref, gb1_ref, gw2_ref, gb2_ref, out_ref):
    x = ag_ref[...] + bg_ref[...]
    d2 = d2_ref[...]
    dl = dl_ref[...]
    st = st_ref[...]
    w1g = w1g_ref[...]
    pre = (x + d2 * w1g[0:1, :] + dl * w1g[1:2, :]
           + jnp.dot(st, w1g[2:10, :], preferred_element_type=F32)
           + eb1_ref[...])
    h = _silu(pre)
    e = _silu(jnp.dot(h, ew2_ref[...], preferred_element_type=F32) + eb2_ref[...])
    gw1 = gw1_ref[...]
    g1 = (d2 * gw1[0:1, :] + dl * gw1[1:2, :]
          + jnp.dot(st, gw1[2:10, :], preferred_element_type=F32)
          + gb1_ref[...])
    gh = _silu(g1)
    glogit = jnp.sum(gh * gw2_ref[...], axis=-1, keepdims=True) + gb2_ref[...]
    out_ref[...] = e * _sigmoid(glogit)


def _edge(ag, bg, d2, dl, st, w1g, eb1, ew2, eb2, gw1, gb1, gw2, gb2):
    e = ag.shape[0]
    blk = 4000
    wspec = lambda shape: pl.BlockSpec(shape, lambda i: tuple(0 for _ in shape))
    return pl.pallas_call(
        _edge_body,
        grid=(e // blk,),
        in_specs=[
            pl.BlockSpec((blk, 64), lambda i: (i, 0)),
            pl.BlockSpec((blk, 64), lambda i: (i, 0)),
            pl.BlockSpec((blk, 1), lambda i: (i, 0)),
            pl.BlockSpec((blk, 1), lambda i: (i, 0)),
            pl.BlockSpec((blk, 8), lambda i: (i, 0)),
            wspec((10, 64)), wspec((1, 64)), wspec((64, 64)), wspec((1, 64)),
            wspec((10, 32)), wspec((1, 32)), wspec((1, 32)), wspec((1, 1)),
        ],
        out_specs=pl.BlockSpec((blk, 64), lambda i: (i, 0)),
        out_shape=jax.ShapeDtypeStruct((e, 64), F32),
    )(ag, bg, d2, dl, st, w1g, eb1, ew2, eb2, gw1, gb1, gw2, gb2)


def _scatter_body(e, n, msg_hbm, ii_hbm, zero_hbm, out_hbm,
                  idx_v, idx_t, rows_v, rows_t, agg_sh):
    c = lax.axis_index("c")
    s = lax.axis_index("s")
    wid = s * _NC + c
    ep = e // _NW
    nfull = ep // _CH
    tail = ep - nfull * _CH
    base = pl.multiple_of(wid * ep, _CH)
    npart = n // _NS

    pltpu.sync_copy(zero_hbm.at[pl.ds(s * npart, npart)],
                    agg_sh.at[pl.ds(s * npart, npart)])
    plsc.subcore_barrier()

    def chunk(k, _):
        off = pl.multiple_of(k * _CH, _CH)
        pltpu.sync_copy(msg_hbm.at[pl.ds(base + off, _CH)], rows_v)
        pltpu.sync_copy(ii_hbm.at[pl.ds(base + off, _CH)], idx_v)
        pltpu.sync_copy(rows_v, agg_sh.at[idx_v], add=True)
        return 0

    lax.fori_loop(0, nfull, chunk, 0)

    if tail:
        off = nfull * _CH
        pltpu.sync_copy(msg_hbm.at[pl.ds(base + off, tail)], rows_t)
        pltpu.sync_copy(ii_hbm.at[pl.ds(base + off, tail)], idx_t)
        pltpu.sync_copy(rows_t, agg_sh.at[idx_t], add=True)

    plsc.subcore_barrier()
    pltpu.sync_copy(agg_sh.at[pl.ds(s * npart, npart)],
                    out_hbm.at[pl.ds(c * n + s * npart, npart)])


def _scatter(msg, ii, zero):
    e, d = msg.shape
    n = zero.shape[0]
    ep = e // _NW
    tail = ep - (ep // _CH) * _CH
    mesh = plsc.VectorSubcoreMesh(core_axis_name="c", subcore_axis_name="s",
                                  num_cores=_NC, num_subcores=_NS)
    k = pl.kernel(
        functools.partial(_scatter_body, e, n),
        mesh=mesh,
        compiler_params=pltpu.CompilerParams(use_tc_tiling_on_sc=False),
        out_type=jax.ShapeDtypeStruct((_NC * n, d), F32),
        scratch_types=[
            pltpu.VMEM((_CH,), jnp.int32),
            pltpu.VMEM((max(tail, 8),), jnp.int32),
            pltpu.VMEM((_CH, d), F32),
            pltpu.VMEM((max(tail, 8), d), F32),
            pltpu.VMEM_SHARED((n, d), F32),
        ],
    )
    return k(msg, ii, zero)


def _node_body(h_ref, agg2_ref, w1a_ref, w1b_ref, nb1_ref, w2_ref, nb2_ref,
               g_ref, b_ref, out_ref):
    h = h_ref[...]
    agg = agg2_ref[0] + agg2_ref[1]
    m1 = (jnp.dot(h, w1a_ref[...], preferred_element_type=F32)
          + jnp.dot(agg, w1b_ref[...], preferred_element_type=F32)
          + nb1_ref[...])
    hm = _silu(m1)
    m = jnp.dot(hm, w2_ref[...], preferred_element_type=F32) + nb2_ref[...]
    y = h + m
    mu = jnp.mean(y, axis=-1, keepdims=True)
    yc = y - mu
    var = jnp.mean(yc * yc, axis=-1, keepdims=True)
    out_ref[...] = yc * lax.rsqrt(var + 1e-5) * g_ref[...] + b_ref[...]


def _node(h, agg2, w1a, w1b, nb1, w2, nb2, g, b):
    n, d = h.shape
    blk = 2000
    wspec = lambda shape: pl.BlockSpec(shape, lambda i: tuple(0 for _ in shape))
    return pl.pallas_call(
        _node_body,
        grid=(n // blk,),
        in_specs=[
            pl.BlockSpec((blk, d), lambda i: (i, 0)),
            pl.BlockSpec((2, blk, 64), lambda i: (0, i, 0)),
            wspec((d, 2 * d)), wspec((64, 2 * d)), wspec((1, 2 * d)),
            wspec((2 * d, d)), wspec((1, d)),
            wspec((1, d)), wspec((1, d)),
        ],
        out_specs=pl.BlockSpec((blk, d), lambda i: (i, 0)),
        out_shape=jax.ShapeDtypeStruct((n, d), F32),
    )(h, agg2, w1a, w1b, nb1, w2, nb2, g, b)


def kernel(H, edge_index, dist2, delta, edge_struct,
           eW1, eb1, eW2, eb2, gW1, gb1, gW2, gb2,
           nW1, nb1, nW2, nb2, ln_g, ln_b):
    bz, n, d = H.shape
    e = edge_index.shape[1]
    d_struct = edge_struct.shape[-1]
    assert bz == 1 and e % _NW == 0 and d == 128

    h0 = H.reshape(n, d)
    ii = edge_index[0]
    jj = edge_index[1]
    d2 = dist2.reshape(e, 1)
    dl = delta.reshape(e, 1)
    st = edge_struct.reshape(e, d_struct)

    wi = eW1[0:d]
    wj = eW1[d:2 * d]
    w1g = eW1[2 * d:]

    a, bm = _proj(h0, wi, wj)
    ag, bg = _gather(a, bm, ii, jj)
    emsg = _edge(ag, bg, d2, dl, st, w1g,
                 eb1.reshape(1, -1), eW2, eb2.reshape(1, -1),
                 gW1, gb1.reshape(1, -1), gW2.reshape(1, -1), gb2.reshape(1, 1))
    zero = jnp.zeros((n, 64), F32)
    agg2 = _scatter(emsg, ii, zero).reshape(_NC, n, 64)
    out = _node(h0, agg2,
                nW1[0:d], nW1[d:], nb1.reshape(1, -1),
                nW2, nb2.reshape(1, -1),
                ln_g.reshape(1, -1), ln_b.reshape(1, -1))
    return out.reshape(bz, n, d)
